# Initial kernel scaffold; baseline (speedup 1.0000x reference)
import jax
import jax.numpy as jnp
from jax.experimental import pallas as pl


def kernel(user, movie, movies_rated_by_this_user, users_who_rated_this_movie, sqrt_of_number_of_movies_rated_by_this_user, sqrt_of_number_of_users_who_rated_this_movie, is_known_user, is_known_movie, Bu, Bi, P, Q, Y):
    raise NotImplementedError("write your pallas kernel here")


# speedup vs baseline: 1.0602x; 1.0602x over previous
import functools

import jax
import jax.numpy as jnp
from jax import lax
from jax.experimental import pallas as pl
from jax.experimental.pallas import tpu as pltpu
from jax.experimental.pallas import tpu_sc as plsc

B = 4096
E = 200
H = 50
L = 16
NC, NS = 2, 16
NW = NC * NS
BPW = B // NW
NFULL = 12
TAIL = E - L
GM = 3.5


def _splat1(ref, i):
    return plsc.load_gather(ref, [jnp.full((L,), i, jnp.int32)])


def _splat2(ref, i):
    idx = jnp.full((L,), i, jnp.int32)
    return plsc.load_gather(ref, [idx, jnp.zeros((L,), jnp.int32)])


def _svdpp_sc(user, movie, mr, sq, iku, ikm, Bu, Bi, P, Q, Y):
    mesh = plsc.VectorSubcoreMesh(core_axis_name="c", subcore_axis_name="s")
    cp = pltpu.CompilerParams(
        needs_layout_passes=False, use_tc_tiling_on_sc=False
    )

    @functools.partial(
        pl.kernel,
        out_type=jax.ShapeDtypeStruct((B,), jnp.float32),
        mesh=mesh,
        compiler_params=cp,
        scratch_types=[
            pltpu.VMEM((BPW,), jnp.int32),
            pltpu.VMEM((BPW,), jnp.int32),
            pltpu.VMEM((BPW, H), jnp.int32),
            pltpu.VMEM((BPW,), jnp.float32),
            pltpu.VMEM((BPW,), jnp.float32),
            pltpu.VMEM((BPW,), jnp.float32),
            pltpu.VMEM((BPW, 1), jnp.float32),
            pltpu.VMEM((BPW, 1), jnp.float32),
            pltpu.VMEM((BPW, E), jnp.float32),
            pltpu.VMEM((BPW, E), jnp.float32),
            pltpu.VMEM((H, E), jnp.float32),
            pltpu.VMEM((H, E), jnp.float32),
            pltpu.VMEM((BPW,), jnp.float32),
            pltpu.SemaphoreType.DMA,
            pltpu.SemaphoreType.DMA,
            pltpu.SemaphoreType.DMA,
        ],
    )
    def kern(user_h, movie_h, mr_h, sq_h, iku_h, ikm_h, bu_h, bi_h, p_h,
             q_h, y_h, out_h, uidx, midx, mr_v, sq_v, iku_v, ikm_v, bu_v,
             bi_v, p_v, q_v, ybuf0, ybuf1, outv, sem_pre, sem0, sem1):
        wid = lax.axis_index("s") * NC + lax.axis_index("c")
        base = wid * BPW

        pltpu.sync_copy(user_h.at[pl.ds(base, BPW)], uidx)
        pltpu.sync_copy(movie_h.at[pl.ds(base, BPW)], midx)
        pltpu.sync_copy(mr_h.at[pl.ds(base, BPW)], mr_v)
        pltpu.sync_copy(sq_h.at[pl.ds(base, BPW)], sq_v)
        pltpu.sync_copy(iku_h.at[pl.ds(base, BPW)], iku_v)
        pltpu.sync_copy(ikm_h.at[pl.ds(base, BPW)], ikm_v)

        hp = pltpu.async_copy(p_h.at[uidx], p_v, sem_pre)
        hq = pltpu.async_copy(q_h.at[midx], q_v, sem_pre)
        hbu = pltpu.async_copy(bu_h.at[uidx], bu_v, sem_pre)
        hbi = pltpu.async_copy(bi_h.at[midx], bi_v, sem_pre)

        pltpu.async_copy(y_h.at[mr_v.at[0]], ybuf0, sem0)

        hp.wait()
        hq.wait()
        hbu.wait()
        hbi.wait()

        lane = lax.iota(jnp.int32, L)
        tail_mask = lane >= (L - (E - NFULL * L))
        zero = jnp.zeros((L,), jnp.float32)

        def compute(b, ybuf):
            def row_body(h, accs):
                out = tuple(
                    accs[j] + ybuf[h, pl.ds(16 * j, L)] for j in range(NFULL)
                ) + (accs[NFULL] + ybuf[h, pl.ds(TAIL, L)],)
                return out

            accs = lax.fori_loop(0, H, row_body, (zero,) * (NFULL + 1))

            iku_s = _splat1(iku_v, b)
            ikm_s = _splat1(ikm_v, b)
            sq_s = _splat1(sq_v, b)
            ysc = iku_s / sq_s
            tsum = zero
            for j in range(NFULL):
                pj = p_v[b, pl.ds(16 * j, L)]
                qj = q_v[b, pl.ds(16 * j, L)]
                tsum = tsum + qj * (pj * iku_s + accs[j] * ysc)
            pt = p_v[b, pl.ds(TAIL, L)]
            qt = q_v[b, pl.ds(TAIL, L)]
            tt = qt * (pt * iku_s + accs[NFULL] * ysc)
            tsum = tsum + jnp.where(tail_mask, tt, zero)
            dot = jnp.sum(tsum)
            bu_s = _splat2(bu_v, b)
            bi_s = _splat2(bi_v, b)
            r = ikm_s * jnp.full((L,), dot, jnp.float32) \
                + bi_s * ikm_s + bu_s * iku_s + GM
            plsc.store_scatter(
                outv, [jnp.full((L,), b, jnp.int32)], r, mask=(lane == 0)
            )

        @pl.loop(0, BPW, step=2)
        def _(g):
            pltpu.async_copy(y_h.at[mr_v.at[g + 1]], ybuf1, sem1)
            pltpu.make_async_copy(y_h.at[mr_v.at[g]], ybuf0, sem0).wait()
            compute(g, ybuf0)

            @pl.when(g + 2 < BPW)
            def _():
                pltpu.async_copy(y_h.at[mr_v.at[g + 2]], ybuf0, sem0)

            pltpu.make_async_copy(y_h.at[mr_v.at[g + 1]], ybuf1, sem1).wait()
            compute(g + 1, ybuf1)

        pltpu.sync_copy(outv, out_h.at[pl.ds(base, BPW)])

    return kern(user, movie, mr, sq, iku, ikm, Bu, Bi, P, Q, Y)


@jax.jit
def kernel(user, movie, movies_rated_by_this_user, users_who_rated_this_movie,
           sqrt_of_number_of_movies_rated_by_this_user,
           sqrt_of_number_of_users_who_rated_this_movie,
           is_known_user, is_known_movie, Bu, Bi, P, Q, Y):
    del users_who_rated_this_movie, sqrt_of_number_of_users_who_rated_this_movie
    sq = sqrt_of_number_of_movies_rated_by_this_user.reshape(B)
    iku = is_known_user.reshape(B)
    ikm = is_known_movie.reshape(B)
    return _svdpp_sc(
        user.astype(jnp.int32), movie.astype(jnp.int32),
        movies_rated_by_this_user.astype(jnp.int32),
        sq, iku, ikm, Bu, Bi, P, Q, Y,
    )
